# initial kernel scaffold (unmeasured)
import jax
import jax.numpy as jnp
from jax import lax
from jax.experimental import pallas as pl
from jax.experimental.pallas import tpu as pltpu

N_DEV = 16
B, Sq, Hq, Dh = 2, 128, 4, 64
SKV_LOC = 128
BH = B * Hq
ROWS = BH * Sq
D_MODEL = 512
NEG = -1e9


def kernel(x, Wq, K_ext, V_ext, Wo):
    def body(x_ref, wq_ref, k_ref, v_ref, wo_ref, out_ref,
             comm_acc, comm_ml, acc_ssem, acc_rsem, ml_ssem, ml_rsem,
             racc, rml):
        my = lax.axis_index("i")
        left = (my + N_DEV - 1) % N_DEV
        right = (my + 1) % N_DEV

        bsem = pltpu.get_barrier_semaphore()
        for nbr in (left, right):
            pl.semaphore_signal(bsem, inc=1, device_id=(nbr,),
                                device_id_type=pl.DeviceIdType.MESH)
        pl.semaphore_wait(bsem, 2)

        q_blk = lax.broadcasted_iota(jnp.int32, (Sq, SKV_LOC), 0) // 64
        k_blk = my * 2 + lax.broadcasted_iota(jnp.int32, (Sq, SKV_LOC), 1) // 64
        mask = (k_blk % 4) == q_blk

        for b in range(B):
            q_full = jnp.dot(x_ref[b], wq_ref[...],
                             preferred_element_type=jnp.float32)
            for h in range(Hq):
                bh = b * Hq + h
                q_bh = q_full[:, h * Dh:(h + 1) * Dh]
                k_bh = k_ref[b, :, h, :]
                s = lax.dot_general(
                    q_bh, k_bh, (((1,), (1,)), ((), ())),
                    preferred_element_type=jnp.float32) * 0.125
                s = jnp.where(mask, s, NEG)
                m = jnp.max(s, axis=1)
                e = jnp.where(mask, jnp.exp(s - m[:, None]), 0.0)
                lsum = jnp.sum(e, axis=1)
                a = jnp.dot(e, v_ref[b, :, h, :],
                            preferred_element_type=jnp.float32)
                comm_ml[0, 0, bh, :] = m
                comm_ml[0, 1, bh, :] = lsum
                rml[0, bh, :] = m
                rml[1, bh, :] = lsum
                comm_acc[0, bh * Sq:(bh + 1) * Sq, :] = a
                racc[bh * Sq:(bh + 1) * Sq, :] = a

        for hp in range(N_DEV - 1):
            rd_a = pltpu.make_async_remote_copy(
                src_ref=comm_acc.at[hp],
                dst_ref=comm_acc.at[hp + 1],
                send_sem=acc_ssem.at[hp],
                recv_sem=acc_rsem.at[hp],
                device_id=(right,),
                device_id_type=pl.DeviceIdType.MESH,
            )
            rd_m = pltpu.make_async_remote_copy(
                src_ref=comm_ml.at[hp],
                dst_ref=comm_ml.at[hp + 1],
                send_sem=ml_ssem.at[hp],
                recv_sem=ml_rsem.at[hp],
                device_id=(right,),
                device_id_type=pl.DeviceIdType.MESH,
            )
            rd_a.start()
            rd_m.start()
            rd_a.wait()
            rd_m.wait()

            sl = hp + 1
            m_in = comm_ml[sl, 0]
            l_in = comm_ml[sl, 1]
            m_old = rml[0]
            l_old = rml[1]
            mx = jnp.maximum(m_old, m_in)
            a_old = jnp.exp(m_old - mx)
            a_in = jnp.exp(m_in - mx)
            rml[0] = mx
            rml[1] = l_old * a_old + l_in * a_in
            racc[...] = (racc[...] * a_old.reshape(ROWS, 1)
                         + comm_acc[sl] * a_in.reshape(ROWS, 1))

        for b in range(B):
            ob = jnp.zeros((Sq, D_MODEL), jnp.float32)
            for h in range(Hq):
                bh = b * Hq + h
                lsum = rml[1, bh, :]
                ctx = racc[bh * Sq:(bh + 1) * Sq, :] / lsum[:, None]
                ob = ob + jnp.dot(ctx, wo_ref[h * Dh:(h + 1) * Dh, :],
                                  preferred_element_type=jnp.float32)
            out_ref[b] = ob

    return pl.pallas_call(
        body,
        out_shape=jax.ShapeDtypeStruct((B, Sq, D_MODEL), jnp.float32),
        in_specs=[pl.BlockSpec(memory_space=pltpu.VMEM)] * 5,
        out_specs=pl.BlockSpec(memory_space=pltpu.VMEM),
        scratch_shapes=[
            pltpu.VMEM((N_DEV, ROWS, Dh), jnp.float32),
            pltpu.VMEM((N_DEV, 2, BH, Sq), jnp.float32),
            pltpu.SemaphoreType.DMA((N_DEV - 1,)),
            pltpu.SemaphoreType.DMA((N_DEV - 1,)),
            pltpu.SemaphoreType.DMA((N_DEV - 1,)),
            pltpu.SemaphoreType.DMA((N_DEV - 1,)),
            pltpu.VMEM((ROWS, Dh), jnp.float32),
            pltpu.VMEM((2, BH, Sq), jnp.float32),
        ],
        compiler_params=pltpu.CompilerParams(collective_id=0),
    )(x, Wq, K_ext, V_ext, Wo)


# baseline (device time: 127809 ns/iter reference)
import jax
import jax.numpy as jnp
from jax import lax
from jax.experimental import pallas as pl
from jax.experimental.pallas import tpu as pltpu

N_DEV = 16
B, Sq, Hq, Dh = 2, 128, 4, 64
SKV_LOC = 128
BH = B * Hq
ROWS = BH * Sq
D_MODEL = 512
NEG = -1e9


def kernel(x, Wq, K_ext, V_ext, Wo):
    def body(x_ref, wq_ref, k_ref, v_ref, wo_ref, out_ref,
             comm_acc, comm_ml, acc_ssem, acc_rsem, ml_ssem, ml_rsem,
             racc, rml):
        my = lax.axis_index("i")
        left = (my + N_DEV - 1) % N_DEV
        right = (my + 1) % N_DEV

        bsem = pltpu.get_barrier_semaphore()
        for nbr in (left, right):
            pl.semaphore_signal(bsem, inc=1, device_id=(nbr,),
                                device_id_type=pl.DeviceIdType.MESH)
        pl.semaphore_wait(bsem, 2)

        q_blk = lax.broadcasted_iota(jnp.int32, (Sq, SKV_LOC), 0) // 64
        k_blk = my * 2 + lax.broadcasted_iota(jnp.int32, (Sq, SKV_LOC), 1) // 64
        mask = (k_blk % 4) == q_blk

        for b in range(B):
            q_full = jnp.dot(x_ref[b], wq_ref[...],
                             preferred_element_type=jnp.float32)
            for h in range(Hq):
                bh = b * Hq + h
                q_bh = q_full[:, h * Dh:(h + 1) * Dh]
                k_bh = k_ref[b, :, h, :]
                s = lax.dot_general(
                    q_bh, k_bh, (((1,), (1,)), ((), ())),
                    preferred_element_type=jnp.float32) * 0.125
                s = jnp.where(mask, s, NEG)
                m = jnp.max(s, axis=1)
                e = jnp.where(mask, jnp.exp(s - m[:, None]), 0.0)
                lsum = jnp.sum(e, axis=1)
                a = jnp.dot(e, v_ref[b, :, h, :],
                            preferred_element_type=jnp.float32)
                comm_ml[0, 0, bh, :] = m
                comm_ml[0, 1, bh, :] = lsum
                rml[0, bh, :] = m
                rml[1, bh, :] = lsum
                comm_acc[0, bh * Sq:(bh + 1) * Sq, :] = a
                racc[bh * Sq:(bh + 1) * Sq, :] = a

        for hp in range(N_DEV - 1):
            rd_a = pltpu.make_async_remote_copy(
                src_ref=comm_acc.at[hp],
                dst_ref=comm_acc.at[hp + 1],
                send_sem=acc_ssem.at[hp],
                recv_sem=acc_rsem.at[hp],
                device_id=(right,),
                device_id_type=pl.DeviceIdType.MESH,
            )
            rd_m = pltpu.make_async_remote_copy(
                src_ref=comm_ml.at[hp],
                dst_ref=comm_ml.at[hp + 1],
                send_sem=ml_ssem.at[hp],
                recv_sem=ml_rsem.at[hp],
                device_id=(right,),
                device_id_type=pl.DeviceIdType.MESH,
            )
            rd_a.start()
            rd_m.start()
            rd_a.wait()
            rd_m.wait()

            sl = hp + 1
            m_in = comm_ml[sl, 0]
            l_in = comm_ml[sl, 1]
            m_old = rml[0]
            l_old = rml[1]
            mx = jnp.maximum(m_old, m_in)
            a_old = jnp.exp(m_old - mx)
            a_in = jnp.exp(m_in - mx)
            rml[0] = mx
            rml[1] = l_old * a_old + l_in * a_in
            for bh in range(BH):
                racc[bh * Sq:(bh + 1) * Sq, :] = (
                    racc[bh * Sq:(bh + 1) * Sq, :] * a_old[bh][:, None]
                    + comm_acc[sl, bh * Sq:(bh + 1) * Sq, :] * a_in[bh][:, None])

        for b in range(B):
            ob = jnp.zeros((Sq, D_MODEL), jnp.float32)
            for h in range(Hq):
                bh = b * Hq + h
                lsum = rml[1, bh, :]
                ctx = racc[bh * Sq:(bh + 1) * Sq, :] / lsum[:, None]
                ob = ob + jnp.dot(ctx, wo_ref[h * Dh:(h + 1) * Dh, :],
                                  preferred_element_type=jnp.float32)
            out_ref[b] = ob

    return pl.pallas_call(
        body,
        out_shape=jax.ShapeDtypeStruct((B, Sq, D_MODEL), jnp.float32),
        in_specs=[pl.BlockSpec(memory_space=pltpu.VMEM)] * 5,
        out_specs=pl.BlockSpec(memory_space=pltpu.VMEM),
        scratch_shapes=[
            pltpu.VMEM((N_DEV, ROWS, Dh), jnp.float32),
            pltpu.VMEM((N_DEV, 2, BH, Sq), jnp.float32),
            pltpu.SemaphoreType.DMA((N_DEV - 1,)),
            pltpu.SemaphoreType.DMA((N_DEV - 1,)),
            pltpu.SemaphoreType.DMA((N_DEV - 1,)),
            pltpu.SemaphoreType.DMA((N_DEV - 1,)),
            pltpu.VMEM((ROWS, Dh), jnp.float32),
            pltpu.VMEM((2, BH, Sq), jnp.float32),
        ],
        compiler_params=pltpu.CompilerParams(collective_id=0),
    )(x, Wq, K_ext, V_ext, Wo)


# device time: 67548 ns/iter; 1.8921x vs baseline; 1.8921x over previous
import functools

import jax
import jax.numpy as jnp
from jax import lax
from jax.experimental import pallas as pl
from jax.experimental.pallas import tpu as pltpu

N_DEV = 16
B, Sq, Hq, Dh = 2, 128, 4, 64
SKV_LOC = 128
BH = B * Hq
ROWS = BH * Sq
D_MODEL = 512
NEG = -1e9
R_HOPS = 4
L_HOPS = 3


def kernel(x, Wq, K_ext, V_ext, Wo):
    def body(x_ref, wq_ref, k_ref, v_ref, wo_ref, out_ref,
             cRa, cRm, cLa, cLm,
             aRs, aRr, mRs, mRr, aLs, aLr, mLs, mLr,
             fas, far, fms, fmr,
             racc, rml, fin_acc, fin_ml):
        my = lax.axis_index("i")
        is_even = (my % 2) == 0
        partner = jnp.where(is_even, my + 1, my - 1)
        right_e = (my + 2) % N_DEV
        left_e = (my + N_DEV - 2) % N_DEV

        bsem = pltpu.get_barrier_semaphore()

        fa = pltpu.make_async_remote_copy(
            src_ref=racc, dst_ref=fin_acc, send_sem=fas, recv_sem=far,
            device_id=(partner,), device_id_type=pl.DeviceIdType.MESH)
        fm = pltpu.make_async_remote_copy(
            src_ref=rml, dst_ref=fin_ml, send_sem=fms, recv_sem=fmr,
            device_id=(partner,), device_id_type=pl.DeviceIdType.MESH)

        def combine(acc_ref, ml_ref, sl):
            m_in = ml_ref[sl, 0]
            l_in = ml_ref[sl, 1]
            m_old = rml[0]
            l_old = rml[1]
            mx = jnp.maximum(m_old, m_in)
            a_old = jnp.exp(m_old - mx)
            a_in = jnp.exp(m_in - mx)
            rml[0] = mx
            rml[1] = l_old * a_old + l_in * a_in
            for bh in range(BH):
                racc[bh * Sq:(bh + 1) * Sq, :] = (
                    racc[bh * Sq:(bh + 1) * Sq, :] * a_old[bh][:, None]
                    + acc_ref[sl, bh * Sq:(bh + 1) * Sq, :] * a_in[bh][:, None])

        @pl.when(is_even)
        def _even():
            for nbr in (left_e, right_e, my + 1):
                pl.semaphore_signal(bsem, inc=1, device_id=(nbr,),
                                    device_id_type=pl.DeviceIdType.MESH)

            q_blk = lax.broadcasted_iota(jnp.int32, (Sq, SKV_LOC), 0) // 64
            k_blk = my * 2 + lax.broadcasted_iota(jnp.int32, (Sq, SKV_LOC), 1) // 64
            mask = (k_blk % 4) == q_blk
            for b in range(B):
                q_full = jnp.dot(x_ref[b], wq_ref[...],
                                 preferred_element_type=jnp.float32)
                for h in range(Hq):
                    bh = b * Hq + h
                    q_bh = q_full[:, h * Dh:(h + 1) * Dh]
                    k_bh = k_ref[b, :, h, :]
                    s = lax.dot_general(
                        q_bh, k_bh, (((1,), (1,)), ((), ())),
                        preferred_element_type=jnp.float32) * 0.125
                    s = jnp.where(mask, s, NEG)
                    m = jnp.max(s, axis=1)
                    e = jnp.where(mask, jnp.exp(s - m[:, None]), 0.0)
                    lsum = jnp.sum(e, axis=1)
                    a = jnp.dot(e, v_ref[b, :, h, :],
                                preferred_element_type=jnp.float32)
                    for mlr in (cRm, cLm):
                        mlr[0, 0, bh, :] = m
                        mlr[0, 1, bh, :] = lsum
                    rml[0, bh, :] = m
                    rml[1, bh, :] = lsum
                    for accr in (cRa, cLa):
                        accr[0, bh * Sq:(bh + 1) * Sq, :] = a
                    racc[bh * Sq:(bh + 1) * Sq, :] = a

            pl.semaphore_wait(bsem, 3)

            def mk(acc_ref, ml_ref, a_s, a_r, m_s, m_r, h, dev):
                ra = pltpu.make_async_remote_copy(
                    src_ref=acc_ref.at[h], dst_ref=acc_ref.at[h + 1],
                    send_sem=a_s.at[h], recv_sem=a_r.at[h],
                    device_id=(dev,), device_id_type=pl.DeviceIdType.MESH)
                rm = pltpu.make_async_remote_copy(
                    src_ref=ml_ref.at[h], dst_ref=ml_ref.at[h + 1],
                    send_sem=m_s.at[h], recv_sem=m_r.at[h],
                    device_id=(dev,), device_id_type=pl.DeviceIdType.MESH)
                ra.start()
                rm.start()
                return ra, rm

            rs = [mk(cRa, cRm, aRs, aRr, mRs, mRr, 0, right_e)]
            ls = [mk(cLa, cLm, aLs, aLr, mLs, mLr, 0, left_e)]
            for h in range(R_HOPS):
                rs[h][0].wait()
                rs[h][1].wait()
                if h < L_HOPS:
                    ls[h][0].wait()
                    ls[h][1].wait()
                if h + 1 < R_HOPS:
                    rs.append(mk(cRa, cRm, aRs, aRr, mRs, mRr, h + 1, right_e))
                if h + 1 < L_HOPS:
                    ls.append(mk(cLa, cLm, aLs, aLr, mLs, mLr, h + 1, left_e))
                combine(cRa, cRm, h + 1)
                if h < L_HOPS:
                    combine(cLa, cLm, h + 1)

            fa.start()
            fm.start()
            fin_acc[...] = racc[...]
            fin_ml[...] = rml[...]

        @pl.when(jnp.logical_not(is_even))
        def _odd():
            pl.semaphore_signal(bsem, inc=1, device_id=(partner,),
                                device_id_type=pl.DeviceIdType.MESH)
            pl.semaphore_wait(bsem, 1)
            fa.wait_recv()
            fm.wait_recv()

        for b in range(B):
            ob = jnp.zeros((Sq, D_MODEL), jnp.float32)
            for h in range(Hq):
                bh = b * Hq + h
                lsum = fin_ml[1, bh, :]
                ctx = fin_acc[bh * Sq:(bh + 1) * Sq, :] / lsum[:, None]
                ob = ob + jnp.dot(ctx, wo_ref[h * Dh:(h + 1) * Dh, :],
                                  preferred_element_type=jnp.float32)
            out_ref[b] = ob

        @pl.when(is_even)
        def _even_drain():
            fa.wait_send()
            fm.wait_send()

        @functools.partial(pl.run_scoped, ack=pltpu.SemaphoreType.REGULAR)
        def _(ack):
            @pl.when(jnp.logical_not(is_even))
            def _():
                pl.semaphore_signal(ack, inc=1, device_id=(partner,),
                                    device_id_type=pl.DeviceIdType.MESH)

            @pl.when(is_even)
            def _():
                pl.semaphore_wait(ack, 1)

    return pl.pallas_call(
        body,
        out_shape=jax.ShapeDtypeStruct((B, Sq, D_MODEL), jnp.float32),
        in_specs=[pl.BlockSpec(memory_space=pltpu.VMEM)] * 5,
        out_specs=pl.BlockSpec(memory_space=pltpu.VMEM),
        scratch_shapes=[
            pltpu.VMEM((R_HOPS + 1, ROWS, Dh), jnp.float32),
            pltpu.VMEM((R_HOPS + 1, 2, BH, Sq), jnp.float32),
            pltpu.VMEM((L_HOPS + 1, ROWS, Dh), jnp.float32),
            pltpu.VMEM((L_HOPS + 1, 2, BH, Sq), jnp.float32),
            pltpu.SemaphoreType.DMA((R_HOPS,)),
            pltpu.SemaphoreType.DMA((R_HOPS,)),
            pltpu.SemaphoreType.DMA((R_HOPS,)),
            pltpu.SemaphoreType.DMA((R_HOPS,)),
            pltpu.SemaphoreType.DMA((L_HOPS,)),
            pltpu.SemaphoreType.DMA((L_HOPS,)),
            pltpu.SemaphoreType.DMA((L_HOPS,)),
            pltpu.SemaphoreType.DMA((L_HOPS,)),
            pltpu.SemaphoreType.DMA,
            pltpu.SemaphoreType.DMA,
            pltpu.SemaphoreType.DMA,
            pltpu.SemaphoreType.DMA,
            pltpu.VMEM((ROWS, Dh), jnp.float32),
            pltpu.VMEM((2, BH, Sq), jnp.float32),
            pltpu.VMEM((ROWS, Dh), jnp.float32),
            pltpu.VMEM((2, BH, Sq), jnp.float32),
        ],
        compiler_params=pltpu.CompilerParams(collective_id=0),
    )(x, Wq, K_ext, V_ext, Wo)


# device time: 34687 ns/iter; 3.6846x vs baseline; 1.9474x over previous
import functools

import jax
import jax.numpy as jnp
from jax import lax
from jax.experimental import pallas as pl
from jax.experimental.pallas import tpu as pltpu

N_DEV = 16
B, Sq, Hq, Dh = 2, 128, 4, 64
SKV_LOC = 128
BH = B * Hq
ROWS = BH * Sq
D_MODEL = 512
NEG = -1e9
R_HOPS = 4
L_HOPS = 3


def kernel(x, Wq, K_ext, V_ext, Wo):
    def body(x_ref, wq_ref, k_ref, v_ref, wo_ref, out_ref,
             cRa, cRm, cLa, cLm,
             aRs, aRr, mRs, mRr, aLs, aLr, mLs, mLr,
             fas, far, fms, fmr,
             racc, rml, fin_acc, fin_ml):
        my = lax.axis_index("i")
        is_even = (my % 2) == 0
        partner = jnp.where(is_even, my + 1, my - 1)

        k_idx = jnp.where(my % 4 == 0, my // 4, 4 + (14 - my) // 4)

        def cyc(kk):
            return jnp.where(kk <= 3, 4 * kk, 30 - 4 * kk)

        right_e = cyc((k_idx + 1) % 8)
        left_e = cyc((k_idx + 7) % 8)

        bsem = pltpu.get_barrier_semaphore()

        fa = pltpu.make_async_remote_copy(
            src_ref=fin_acc, dst_ref=fin_acc, send_sem=fas, recv_sem=far,
            device_id=(partner,), device_id_type=pl.DeviceIdType.MESH)
        fm = pltpu.make_async_remote_copy(
            src_ref=fin_ml, dst_ref=fin_ml, send_sem=fms, recv_sem=fmr,
            device_id=(partner,), device_id_type=pl.DeviceIdType.MESH)

        def combine(acc_ref, ml_ref, sl):
            m_in = ml_ref[sl, 0]
            l_in = ml_ref[sl, 1]
            m_old = rml[0]
            l_old = rml[1]
            mx = jnp.maximum(m_old, m_in)
            a_old = jnp.exp(m_old - mx)
            a_in = jnp.exp(m_in - mx)
            rml[0] = mx
            rml[1] = l_old * a_old + l_in * a_in
            for bh in range(BH):
                racc[bh * Sq:(bh + 1) * Sq, :] = (
                    racc[bh * Sq:(bh + 1) * Sq, :] * a_old[bh][:, None]
                    + acc_ref[sl, bh * Sq:(bh + 1) * Sq, :].astype(jnp.float32)
                    * a_in[bh][:, None])

        @pl.when(is_even)
        def _even():
            for nbr in (left_e, right_e, my + 1):
                pl.semaphore_signal(bsem, inc=1, device_id=(nbr,),
                                    device_id_type=pl.DeviceIdType.MESH)

            q_blk = lax.broadcasted_iota(jnp.int32, (Sq, SKV_LOC), 0) // 64
            k_blk = my * 2 + lax.broadcasted_iota(jnp.int32, (Sq, SKV_LOC), 1) // 64
            mask = (k_blk % 4) == q_blk
            for b in range(B):
                q_full = jnp.dot(x_ref[b], wq_ref[...],
                                 preferred_element_type=jnp.float32)
                for h in range(Hq):
                    bh = b * Hq + h
                    q_bh = q_full[:, h * Dh:(h + 1) * Dh]
                    k_bh = k_ref[b, :, h, :]
                    s = lax.dot_general(
                        q_bh, k_bh, (((1,), (1,)), ((), ())),
                        preferred_element_type=jnp.float32) * 0.125
                    s = jnp.where(mask, s, NEG)
                    m = jnp.max(s, axis=1)
                    e = jnp.where(mask, jnp.exp(s - m[:, None]), 0.0)
                    lsum = jnp.sum(e, axis=1)
                    a = jnp.dot(e, v_ref[b, :, h, :],
                                preferred_element_type=jnp.float32)
                    for mlr in (cRm, cLm):
                        mlr[0, 0, bh, :] = m
                        mlr[0, 1, bh, :] = lsum
                    rml[0, bh, :] = m
                    rml[1, bh, :] = lsum
                    a_bf = a.astype(jnp.bfloat16)
                    for accr in (cRa, cLa):
                        accr[0, bh * Sq:(bh + 1) * Sq, :] = a_bf
                    racc[bh * Sq:(bh + 1) * Sq, :] = a

            pl.semaphore_wait(bsem, 3)

            def mk(acc_ref, ml_ref, a_s, a_r, m_s, m_r, h, dev):
                ra = pltpu.make_async_remote_copy(
                    src_ref=acc_ref.at[h], dst_ref=acc_ref.at[h + 1],
                    send_sem=a_s.at[h], recv_sem=a_r.at[h],
                    device_id=(dev,), device_id_type=pl.DeviceIdType.MESH)
                rm = pltpu.make_async_remote_copy(
                    src_ref=ml_ref.at[h], dst_ref=ml_ref.at[h + 1],
                    send_sem=m_s.at[h], recv_sem=m_r.at[h],
                    device_id=(dev,), device_id_type=pl.DeviceIdType.MESH)
                ra.start()
                rm.start()
                return ra, rm

            rs = [mk(cRa, cRm, aRs, aRr, mRs, mRr, 0, right_e)]
            ls = [mk(cLa, cLm, aLs, aLr, mLs, mLr, 0, left_e)]
            for h in range(R_HOPS):
                rs[h][0].wait_recv()
                rs[h][1].wait_recv()
                if h < L_HOPS:
                    ls[h][0].wait_recv()
                    ls[h][1].wait_recv()
                if h + 1 < R_HOPS:
                    rs.append(mk(cRa, cRm, aRs, aRr, mRs, mRr, h + 1, right_e))
                if h + 1 < L_HOPS:
                    ls.append(mk(cLa, cLm, aLs, aLr, mLs, mLr, h + 1, left_e))
                combine(cRa, cRm, h + 1)
                if h < L_HOPS:
                    combine(cLa, cLm, h + 1)

            for bh in range(BH):
                fin_acc[bh * Sq:(bh + 1) * Sq, :] = (
                    racc[bh * Sq:(bh + 1) * Sq, :].astype(jnp.bfloat16))
            fin_ml[...] = rml[...]
            fa.start()
            fm.start()

            for ra, rm in rs + ls:
                ra.wait_send()
                rm.wait_send()

        @pl.when(jnp.logical_not(is_even))
        def _odd():
            pl.semaphore_signal(bsem, inc=1, device_id=(partner,),
                                device_id_type=pl.DeviceIdType.MESH)
            pl.semaphore_wait(bsem, 1)
            fa.wait_recv()
            fm.wait_recv()

        for b in range(B):
            ob = jnp.zeros((Sq, D_MODEL), jnp.float32)
            for h in range(Hq):
                bh = b * Hq + h
                lsum = fin_ml[1, bh, :]
                ctx = (fin_acc[bh * Sq:(bh + 1) * Sq, :].astype(jnp.float32)
                       / lsum[:, None])
                ob = ob + jnp.dot(ctx, wo_ref[h * Dh:(h + 1) * Dh, :],
                                  preferred_element_type=jnp.float32)
            out_ref[b] = ob

        @pl.when(is_even)
        def _even_drain():
            fa.wait_send()
            fm.wait_send()

        @functools.partial(pl.run_scoped, ack=pltpu.SemaphoreType.REGULAR)
        def _(ack):
            @pl.when(jnp.logical_not(is_even))
            def _():
                pl.semaphore_signal(ack, inc=1, device_id=(partner,),
                                    device_id_type=pl.DeviceIdType.MESH)

            @pl.when(is_even)
            def _():
                pl.semaphore_wait(ack, 1)

    return pl.pallas_call(
        body,
        out_shape=jax.ShapeDtypeStruct((B, Sq, D_MODEL), jnp.float32),
        in_specs=[pl.BlockSpec(memory_space=pltpu.VMEM)] * 5,
        out_specs=pl.BlockSpec(memory_space=pltpu.VMEM),
        scratch_shapes=[
            pltpu.VMEM((R_HOPS + 1, ROWS, Dh), jnp.bfloat16),
            pltpu.VMEM((R_HOPS + 1, 2, BH, Sq), jnp.float32),
            pltpu.VMEM((L_HOPS + 1, ROWS, Dh), jnp.bfloat16),
            pltpu.VMEM((L_HOPS + 1, 2, BH, Sq), jnp.float32),
            pltpu.SemaphoreType.DMA((R_HOPS,)),
            pltpu.SemaphoreType.DMA((R_HOPS,)),
            pltpu.SemaphoreType.DMA((R_HOPS,)),
            pltpu.SemaphoreType.DMA((R_HOPS,)),
            pltpu.SemaphoreType.DMA((L_HOPS,)),
            pltpu.SemaphoreType.DMA((L_HOPS,)),
            pltpu.SemaphoreType.DMA((L_HOPS,)),
            pltpu.SemaphoreType.DMA((L_HOPS,)),
            pltpu.SemaphoreType.DMA,
            pltpu.SemaphoreType.DMA,
            pltpu.SemaphoreType.DMA,
            pltpu.SemaphoreType.DMA,
            pltpu.VMEM((ROWS, Dh), jnp.float32),
            pltpu.VMEM((2, BH, Sq), jnp.float32),
            pltpu.VMEM((ROWS, Dh), jnp.bfloat16),
            pltpu.VMEM((2, BH, Sq), jnp.float32),
        ],
        compiler_params=pltpu.CompilerParams(collective_id=0),
    )(x, Wq, K_ext, V_ext, Wo)


# device time: 34303 ns/iter; 3.7259x vs baseline; 1.0112x over previous
import functools

import jax
import jax.numpy as jnp
from jax import lax
from jax.experimental import pallas as pl
from jax.experimental.pallas import tpu as pltpu

N_DEV = 16
B, Sq, Hq, Dh = 2, 128, 4, 64
SKV_LOC = 128
BH = B * Hq
ROWS = BH * Sq
D_MODEL = 512
NEG = -1e9
R_HOPS = 4
L_HOPS = 3


def kernel(x, Wq, K_ext, V_ext, Wo):
    def body(x_ref, wq_ref, k_ref, v_ref, wo_ref, out_ref,
             cRa, cRm, cLa, cLm,
             aRs, aRr, mRs, mRr, aLs, aLr, mLs, mLr,
             fas, far, fms, fmr,
             racc, rml, fin_acc, fin_ml):
        my = lax.axis_index("i")
        is_even = (my % 2) == 0
        partner = jnp.where(is_even, my + 1, my - 1)

        k_idx = jnp.where(my % 4 == 0, my // 4, 4 + (14 - my) // 4)

        def cyc(kk):
            return jnp.where(kk <= 3, 4 * kk, 30 - 4 * kk)

        right_e = cyc((k_idx + 1) % 8)
        left_e = cyc((k_idx + 7) % 8)

        bsem = pltpu.get_barrier_semaphore()

        fa = pltpu.make_async_remote_copy(
            src_ref=fin_acc, dst_ref=fin_acc, send_sem=fas, recv_sem=far,
            device_id=(partner,), device_id_type=pl.DeviceIdType.MESH)
        fm = pltpu.make_async_remote_copy(
            src_ref=fin_ml, dst_ref=fin_ml, send_sem=fms, recv_sem=fmr,
            device_id=(partner,), device_id_type=pl.DeviceIdType.MESH)

        def combine(acc_ref, ml_ref, sl):
            m_in = ml_ref[sl, 0]
            l_in = ml_ref[sl, 1]
            m_old = rml[0]
            l_old = rml[1]
            mx = jnp.maximum(m_old, m_in)
            a_old = jnp.exp(m_old - mx)
            a_in = jnp.exp(m_in - mx)
            rml[0] = mx
            rml[1] = l_old * a_old + l_in * a_in
            for bh in range(BH):
                racc[bh * Sq:(bh + 1) * Sq, :] = (
                    racc[bh * Sq:(bh + 1) * Sq, :] * a_old[bh][:, None]
                    + acc_ref[sl, bh * Sq:(bh + 1) * Sq, :].astype(jnp.float32)
                    * a_in[bh][:, None])

        @pl.when(is_even)
        def _even():
            for nbr in (left_e, right_e, my + 1):
                pl.semaphore_signal(bsem, inc=1, device_id=(nbr,),
                                    device_id_type=pl.DeviceIdType.MESH)

            q_blk = lax.broadcasted_iota(jnp.int32, (Sq, SKV_LOC), 0) // 64
            k_blk = my * 2 + lax.broadcasted_iota(jnp.int32, (Sq, SKV_LOC), 1) // 64
            mask = (k_blk % 4) == q_blk
            for b in range(B):
                q_full = jnp.dot(x_ref[b], wq_ref[...],
                                 preferred_element_type=jnp.float32)
                for h in range(Hq):
                    bh = b * Hq + h
                    q_bh = q_full[:, h * Dh:(h + 1) * Dh]
                    k_bh = k_ref[b, :, h, :]
                    s = lax.dot_general(
                        q_bh, k_bh, (((1,), (1,)), ((), ())),
                        preferred_element_type=jnp.float32) * 0.125
                    s = jnp.where(mask, s, NEG)
                    m = jnp.max(s, axis=1)
                    e = jnp.where(mask, jnp.exp(s - m[:, None]), 0.0)
                    lsum = jnp.sum(e, axis=1)
                    a = jnp.dot(e, v_ref[b, :, h, :],
                                preferred_element_type=jnp.float32)
                    for mlr in (cRm, cLm):
                        mlr[0, 0, bh, :] = m
                        mlr[0, 1, bh, :] = lsum
                    rml[0, bh, :] = m
                    rml[1, bh, :] = lsum
                    a_bf = a.astype(jnp.bfloat16)
                    for accr in (cRa, cLa):
                        accr[0, bh * Sq:(bh + 1) * Sq, :] = a_bf
                    racc[bh * Sq:(bh + 1) * Sq, :] = a

            pl.semaphore_wait(bsem, 2)

            def mk1(buf, s_s, s_r, h, dev):
                r = pltpu.make_async_remote_copy(
                    src_ref=buf.at[h], dst_ref=buf.at[h + 1],
                    send_sem=s_s.at[h], recv_sem=s_r.at[h],
                    device_id=(dev,), device_id_type=pl.DeviceIdType.MESH)
                r.start()
                return r

            def mk(acc_ref, ml_ref, a_s, a_r, m_s, m_r, h, dev):
                return (mk1(acc_ref, a_s, a_r, h, dev),
                        mk1(ml_ref, m_s, m_r, h, dev))

            rs = [mk(cRa, cRm, aRs, aRr, mRs, mRr, 0, right_e)]
            ls = [mk(cLa, cLm, aLs, aLr, mLs, mLr, 0, left_e)]
            for h in range(R_HOPS):
                rs[h][0].wait_recv()
                if h + 1 < R_HOPS:
                    next_ra = mk1(cRa, aRs, aRr, h + 1, right_e)
                if h < L_HOPS:
                    ls[h][0].wait_recv()
                    if h + 1 < L_HOPS:
                        next_la = mk1(cLa, aLs, aLr, h + 1, left_e)
                rs[h][1].wait_recv()
                if h + 1 < R_HOPS:
                    rs.append((next_ra, mk1(cRm, mRs, mRr, h + 1, right_e)))
                if h < L_HOPS:
                    ls[h][1].wait_recv()
                    if h + 1 < L_HOPS:
                        ls.append((next_la, mk1(cLm, mLs, mLr, h + 1, left_e)))
                combine(cRa, cRm, h + 1)
                if h < L_HOPS:
                    combine(cLa, cLm, h + 1)

            pl.semaphore_wait(bsem, 1)
            for bh in range(BH):
                fin_acc[bh * Sq:(bh + 1) * Sq, :] = (
                    racc[bh * Sq:(bh + 1) * Sq, :].astype(jnp.bfloat16))
            fin_ml[...] = rml[...]
            fa.start()
            fm.start()

            for ra, rm in rs + ls:
                ra.wait_send()
                rm.wait_send()

        @pl.when(jnp.logical_not(is_even))
        def _odd():
            pl.semaphore_signal(bsem, inc=1, device_id=(partner,),
                                device_id_type=pl.DeviceIdType.MESH)
            pl.semaphore_wait(bsem, 1)
            fa.wait_recv()
            fm.wait_recv()

        for b in range(B):
            ob = jnp.zeros((Sq, D_MODEL), jnp.float32)
            for h in range(Hq):
                bh = b * Hq + h
                lsum = fin_ml[1, bh, :]
                ctx = (fin_acc[bh * Sq:(bh + 1) * Sq, :].astype(jnp.float32)
                       / lsum[:, None])
                ob = ob + jnp.dot(ctx, wo_ref[h * Dh:(h + 1) * Dh, :],
                                  preferred_element_type=jnp.float32)
            out_ref[b] = ob

        @pl.when(is_even)
        def _even_drain():
            fa.wait_send()
            fm.wait_send()

        @functools.partial(pl.run_scoped, ack=pltpu.SemaphoreType.REGULAR)
        def _(ack):
            @pl.when(jnp.logical_not(is_even))
            def _():
                pl.semaphore_signal(ack, inc=1, device_id=(partner,),
                                    device_id_type=pl.DeviceIdType.MESH)

            @pl.when(is_even)
            def _():
                pl.semaphore_wait(ack, 1)

    return pl.pallas_call(
        body,
        out_shape=jax.ShapeDtypeStruct((B, Sq, D_MODEL), jnp.float32),
        in_specs=[pl.BlockSpec(memory_space=pltpu.VMEM)] * 5,
        out_specs=pl.BlockSpec(memory_space=pltpu.VMEM),
        scratch_shapes=[
            pltpu.VMEM((R_HOPS + 1, ROWS, Dh), jnp.bfloat16),
            pltpu.VMEM((R_HOPS + 1, 2, BH, Sq), jnp.float32),
            pltpu.VMEM((L_HOPS + 1, ROWS, Dh), jnp.bfloat16),
            pltpu.VMEM((L_HOPS + 1, 2, BH, Sq), jnp.float32),
            pltpu.SemaphoreType.DMA((R_HOPS,)),
            pltpu.SemaphoreType.DMA((R_HOPS,)),
            pltpu.SemaphoreType.DMA((R_HOPS,)),
            pltpu.SemaphoreType.DMA((R_HOPS,)),
            pltpu.SemaphoreType.DMA((L_HOPS,)),
            pltpu.SemaphoreType.DMA((L_HOPS,)),
            pltpu.SemaphoreType.DMA((L_HOPS,)),
            pltpu.SemaphoreType.DMA((L_HOPS,)),
            pltpu.SemaphoreType.DMA,
            pltpu.SemaphoreType.DMA,
            pltpu.SemaphoreType.DMA,
            pltpu.SemaphoreType.DMA,
            pltpu.VMEM((ROWS, Dh), jnp.float32),
            pltpu.VMEM((2, BH, Sq), jnp.float32),
            pltpu.VMEM((ROWS, Dh), jnp.bfloat16),
            pltpu.VMEM((2, BH, Sq), jnp.float32),
        ],
        compiler_params=pltpu.CompilerParams(collective_id=0),
    )(x, Wq, K_ext, V_ext, Wo)
